# Initial kernel scaffold; baseline (speedup 1.0000x reference)
import jax
import jax.numpy as jnp
from jax.experimental import pallas as pl


def kernel(latents_mean, codebook):
    raise NotImplementedError("write your pallas kernel here")


# speedup vs baseline: 1.0854x; 1.0854x over previous
import jax
import jax.numpy as jnp
from jax.experimental import pallas as pl
from jax.experimental.pallas import tpu as pltpu

_K = 512
_D = 32
_N = 65536
_B = 1024
_BETA = 0.25
_DELTA = 1.0


def _vq_body(x_ref, cb_ref, x2_ref, c2_ref, dist_ref, qst_ref, inds_ref,
             loss_ref, perp_ref, counts_ref, sq_ref):
    i = pl.program_id(0)
    x = x_ref[...]
    cb = cb_ref[...]
    xc = jnp.dot(x, cb.T, preferred_element_type=jnp.float32)
    dist = x2_ref[...] + c2_ref[...] - 2.0 * xc
    dist_ref[...] = dist

    iota = jax.lax.broadcasted_iota(jnp.int32, (_B, _K), 1)
    rowmin = jnp.min(dist, axis=1, keepdims=True)
    inds = jnp.min(jnp.where(dist == rowmin, iota, _K), axis=1)
    inds_ref[...] = inds[:, None]

    one_hot = (iota == inds[:, None]).astype(jnp.float32)
    q = jnp.dot(one_hot, cb, preferred_element_type=jnp.float32)
    qst_ref[...] = x + (q - x)

    @pl.when(i == 0)
    def _init():
        sq_ref[...] = jnp.zeros_like(sq_ref)
        counts_ref[...] = jnp.zeros_like(counts_ref)

    sq_ref[...] = sq_ref[...] + jnp.sum((q - x) ** 2)
    counts_ref[...] = counts_ref[...] + jnp.sum(one_hot, axis=0, keepdims=True)

    @pl.when(i == pl.num_programs(0) - 1)
    def _fin():
        total_sq = sq_ref[0, 0]
        loss_ref[...] = ((_BETA + _DELTA) / (_N * _D) * total_sq).reshape(1, 1)
        p = counts_ref[...] * (1.0 / _N)
        ent = jnp.sum(p * jnp.log(p + 1e-10))
        perp_ref[...] = jnp.exp(-ent).reshape(1, 1)


def kernel(latents_mean, codebook):
    x2 = jnp.sum(latents_mean ** 2, axis=1, keepdims=True)
    c2 = jnp.sum(codebook ** 2, axis=1)[None, :]
    dist, qst, inds, loss, perp = pl.pallas_call(
        _vq_body,
        grid=(_N // _B,),
        in_specs=[
            pl.BlockSpec((_B, _D), lambda i: (i, 0)),
            pl.BlockSpec((_K, _D), lambda i: (0, 0)),
            pl.BlockSpec((_B, 1), lambda i: (i, 0)),
            pl.BlockSpec((1, _K), lambda i: (0, 0)),
        ],
        out_specs=[
            pl.BlockSpec((_B, _K), lambda i: (i, 0)),
            pl.BlockSpec((_B, _D), lambda i: (i, 0)),
            pl.BlockSpec((_B, 1), lambda i: (i, 0)),
            pl.BlockSpec((1, 1), lambda i: (0, 0)),
            pl.BlockSpec((1, 1), lambda i: (0, 0)),
        ],
        out_shape=[
            jax.ShapeDtypeStruct((_N, _K), jnp.float32),
            jax.ShapeDtypeStruct((_N, _D), jnp.float32),
            jax.ShapeDtypeStruct((_N, 1), jnp.int32),
            jax.ShapeDtypeStruct((1, 1), jnp.float32),
            jax.ShapeDtypeStruct((1, 1), jnp.float32),
        ],
        scratch_shapes=[
            pltpu.VMEM((1, _K), jnp.float32),
            pltpu.VMEM((1, 1), jnp.float32),
        ],
    )(latents_mean, codebook, x2, c2)
    return (qst, loss.reshape(()), perp.reshape(()), inds, dist)
